# baseline (speedup 1.0000x reference)
import functools

import jax
import jax.numpy as jnp
from jax import lax
from jax.experimental import pallas as pl
from jax.experimental.pallas import tpu as pltpu
from jax.experimental.pallas import tpu_sc as plsc

DIM = 64
NC = 2
NS = 16
NW = NC * NS
GROUP = 128
LANES = 16


@functools.lru_cache(maxsize=None)
def _make_kernel(B: int):
    bpw = B // NW
    G = bpw // GROUP
    mesh = plsc.VectorSubcoreMesh(core_axis_name="c", subcore_axis_name="s")

    @functools.partial(
        pl.kernel,
        mesh=mesh,
        out_type=jax.ShapeDtypeStruct((B, DIM), jnp.float32),
        scratch_types=[
            pltpu.VMEM((G, GROUP), jnp.int32),
            pltpu.VMEM((GROUP, DIM), jnp.float32),
            pltpu.SemaphoreType.DMA,
        ],
        compiler_params=pltpu.CompilerParams(
            use_tc_tiling_on_sc=False, needs_layout_passes=False
        ),
    )
    def emb(idx_hbm, table_hbm, out_hbm, idx_v, rows_v, sem):
        wid = lax.axis_index("s") * NC + lax.axis_index("c")
        pltpu.sync_copy(idx_hbm.at[wid], idx_v)
        base = wid * bpw

        def group_body(g, carry):
            pltpu.async_copy(table_hbm.at[idx_v.at[g]], rows_v, sem).wait()
            for s in range(GROUP // LANES):
                idxs = idx_v[g, pl.ds(s * LANES, LANES)]
                m0 = idxs == 0
                nz = jnp.sum(m0.astype(jnp.int32))

                @pl.when(nz > 0)
                def _():
                    rows16 = lax.iota(jnp.int32, LANES) + (s * LANES)
                    zeros = jnp.zeros((LANES,), jnp.float32)
                    for c in range(DIM):
                        cols = jnp.full((LANES,), c, jnp.int32)
                        plsc.store_scatter(rows_v, [rows16, cols], zeros, mask=m0)

            pltpu.sync_copy(rows_v, out_hbm.at[pl.ds(base + g * GROUP, GROUP)])
            return carry

        lax.fori_loop(0, G, group_body, 0)

    return emb


def kernel(input, table):
    batch, width = input.shape
    B = batch * width
    idx = input.reshape(NW, (B // NW) // GROUP, GROUP)
    out = _make_kernel(B)(idx, table)
    return out.reshape(batch, width, DIM)


# speedup vs baseline: 1.0138x; 1.0138x over previous
import functools

import jax
import jax.numpy as jnp
from jax import lax
from jax.experimental import pallas as pl
from jax.experimental.pallas import tpu as pltpu
from jax.experimental.pallas import tpu_sc as plsc

DIM = 64
NC = 2
NS = 16
NW = NC * NS
GROUP = 128
GPC = 2
CHUNK = GROUP * GPC
LANES = 16


@functools.lru_cache(maxsize=None)
def _make_kernel(B: int):
    bpw = B // NW
    G = bpw // GROUP
    NCHUNK = G // GPC
    mesh = plsc.VectorSubcoreMesh(core_axis_name="c", subcore_axis_name="s")

    @functools.partial(
        pl.kernel,
        mesh=mesh,
        out_type=jax.ShapeDtypeStruct((B, DIM), jnp.float32),
        scratch_types=[
            pltpu.VMEM((G, GROUP), jnp.int32),
            pltpu.VMEM((2, CHUNK, DIM), jnp.float32),
            pltpu.VMEM((CHUNK,), jnp.int32),
            pltpu.SemaphoreType.DMA((2,)),
            pltpu.SemaphoreType.DMA((2,)),
        ],
        compiler_params=pltpu.CompilerParams(
            use_tc_tiling_on_sc=False, needs_layout_passes=False
        ),
    )
    def emb(idx_hbm, table_hbm, out_hbm, idx_v, rows_v, zpos_v, gsem, ssem):
        wid = lax.axis_index("s") * NC + lax.axis_index("c")
        pltpu.sync_copy(idx_hbm.at[wid], idx_v)
        base = wid * bpw

        def start_gather(c):
            b = c % 2
            for j in range(GPC):
                pltpu.async_copy(
                    table_hbm.at[idx_v.at[c * GPC + j]],
                    rows_v.at[b, pl.ds(j * GROUP, GROUP)],
                    gsem.at[b],
                )

        def wait_gather(c):
            b = c % 2
            for j in range(GPC):
                pltpu.make_async_copy(
                    table_hbm.at[idx_v.at[c * GPC + j]],
                    rows_v.at[b, pl.ds(j * GROUP, GROUP)],
                    gsem.at[b],
                ).wait()

        def fix_zeros(c):
            b = c % 2
            cnt = jnp.int32(0)
            for s in range(CHUNK // LANES):
                idxs = idx_v[c * GPC + s // (GROUP // LANES),
                             pl.ds((s % (GROUP // LANES)) * LANES, LANES)]
                m0 = idxs == 0
                pos16 = lax.iota(jnp.int32, LANES) + (s * LANES)
                plsc.store_compressed(zpos_v.at[pl.ds(cnt, LANES)], pos16, mask=m0)
                cnt = cnt + jnp.sum(m0.astype(jnp.int32))

            zeros = jnp.zeros((LANES,), jnp.float32)

            def zero_one(i, carry):
                r = jnp.max(plsc.load_gather(zpos_v, [jnp.full((LANES,), i, jnp.int32)]))
                for a in range(DIM // LANES):
                    rows_v[b, r, pl.ds(a * LANES, LANES)] = zeros
                return carry

            lax.fori_loop(0, cnt, zero_one, jnp.int32(0))

        def start_store(c):
            b = c % 2
            pltpu.async_copy(
                rows_v.at[b], out_hbm.at[pl.ds(base + c * CHUNK, CHUNK)], ssem.at[b]
            )

        def wait_store(c):
            b = c % 2
            pltpu.make_async_copy(
                rows_v.at[b], out_hbm.at[pl.ds(base + c * CHUNK, CHUNK)], ssem.at[b]
            ).wait()

        start_gather(0)
        for c in range(NCHUNK):
            wait_gather(c)
            fix_zeros(c)
            if c + 1 < NCHUNK:
                if c >= 1:
                    wait_store(c - 1)
                start_gather(c + 1)
            start_store(c)
        if NCHUNK >= 2:
            wait_store(NCHUNK - 2)
        wait_store(NCHUNK - 1)

    return emb


def kernel(input, table):
    batch, width = input.shape
    B = batch * width
    idx = input.reshape(NW, (B // NW) // GROUP, GROUP)
    out = _make_kernel(B)(idx, table)
    return out.reshape(batch, width, DIM)


# speedup vs baseline: 1.0153x; 1.0015x over previous
import functools

import jax
import jax.numpy as jnp
from jax import lax
from jax.experimental import pallas as pl
from jax.experimental.pallas import tpu as pltpu
from jax.experimental.pallas import tpu_sc as plsc

DIM = 64
NC = 2
NS = 16
NW = NC * NS
GROUP = 128
GPC = 2
CHUNK = GROUP * GPC
LANES = 16


@functools.lru_cache(maxsize=None)
def _make_kernel(B: int):
    bpw = B // NW
    G = bpw // GROUP
    NCHUNK = G // GPC
    mesh = plsc.VectorSubcoreMesh(core_axis_name="c", subcore_axis_name="s")

    @functools.partial(
        pl.kernel,
        mesh=mesh,
        out_type=jax.ShapeDtypeStruct((B, DIM), jnp.float32),
        scratch_types=[
            pltpu.VMEM((G, GROUP), jnp.int32),
            pltpu.VMEM((2, CHUNK, DIM), jnp.float32),
            pltpu.VMEM((CHUNK,), jnp.int32),
            pltpu.SemaphoreType.DMA((2,)),
            pltpu.SemaphoreType.DMA((2,)),
        ],
        compiler_params=pltpu.CompilerParams(
            use_tc_tiling_on_sc=False,
            needs_layout_passes=False,
            skip_device_barrier=True,
        ),
    )
    def emb(idx_hbm, table_hbm, out_hbm, idx_v, rows_v, zpos_v, gsem, ssem):
        wid = lax.axis_index("s") * NC + lax.axis_index("c")
        pltpu.sync_copy(idx_hbm.at[wid], idx_v)
        base = wid * bpw

        def start_gather(c):
            b = c % 2
            for j in range(GPC):
                pltpu.async_copy(
                    table_hbm.at[idx_v.at[c * GPC + j]],
                    rows_v.at[b, pl.ds(j * GROUP, GROUP)],
                    gsem.at[b],
                )

        def wait_gather(c):
            b = c % 2
            for j in range(GPC):
                pltpu.make_async_copy(
                    table_hbm.at[idx_v.at[c * GPC + j]],
                    rows_v.at[b, pl.ds(j * GROUP, GROUP)],
                    gsem.at[b],
                ).wait()

        def fix_zeros(c):
            b = c % 2
            cnt = jnp.int32(0)
            for s in range(CHUNK // LANES):
                idxs = idx_v[c * GPC + s // (GROUP // LANES),
                             pl.ds((s % (GROUP // LANES)) * LANES, LANES)]
                m0 = idxs == 0
                pos16 = lax.iota(jnp.int32, LANES) + (s * LANES)
                plsc.store_compressed(zpos_v.at[pl.ds(cnt, LANES)], pos16, mask=m0)
                cnt = cnt + jnp.sum(m0.astype(jnp.int32))

            zeros = jnp.zeros((LANES,), jnp.float32)

            def zero_one(i, carry):
                r = jnp.max(plsc.load_gather(zpos_v, [jnp.full((LANES,), i, jnp.int32)]))
                for a in range(DIM // LANES):
                    rows_v[b, r, pl.ds(a * LANES, LANES)] = zeros
                return carry

            lax.fori_loop(0, cnt, zero_one, jnp.int32(0))

        def start_store(c):
            b = c % 2
            pltpu.async_copy(
                rows_v.at[b], out_hbm.at[pl.ds(base + c * CHUNK, CHUNK)], ssem.at[b]
            )

        def wait_store(c):
            b = c % 2
            pltpu.make_async_copy(
                rows_v.at[b], out_hbm.at[pl.ds(base + c * CHUNK, CHUNK)], ssem.at[b]
            ).wait()

        start_gather(0)
        for c in range(NCHUNK):
            wait_gather(c)
            fix_zeros(c)
            if c + 1 < NCHUNK:
                if c >= 1:
                    wait_store(c - 1)
                start_gather(c + 1)
            start_store(c)
        if NCHUNK >= 2:
            wait_store(NCHUNK - 2)
        wait_store(NCHUNK - 1)

    return emb


def kernel(input, table):
    batch, width = input.shape
    B = batch * width
    idx = input.reshape(NW, (B // NW) // GROUP, GROUP)
    out = _make_kernel(B)(idx, table)
    return out.reshape(batch, width, DIM)


# speedup vs baseline: 1.0285x; 1.0131x over previous
import functools

import jax
import jax.numpy as jnp
from jax import lax
from jax.experimental import pallas as pl
from jax.experimental.pallas import tpu as pltpu
from jax.experimental.pallas import tpu_sc as plsc

DIM = 64
NC = 2
NS = 16
NW = NC * NS
GROUP = 128
GPC = 2
CHUNK = GROUP * GPC
LANES = 16


@functools.lru_cache(maxsize=None)
def _make_kernel(B: int):
    bpw = B // NW
    G = bpw // GROUP
    NCHUNK = G // GPC
    mesh = plsc.VectorSubcoreMesh(core_axis_name="c", subcore_axis_name="s")

    @functools.partial(
        pl.kernel,
        mesh=mesh,
        out_type=jax.ShapeDtypeStruct((B, DIM), jnp.float32),
        scratch_types=[
            pltpu.VMEM((G, GROUP), jnp.int32),
            pltpu.VMEM((2, CHUNK, DIM), jnp.float32),
            pltpu.VMEM((CHUNK,), jnp.int32),
            pltpu.SemaphoreType.DMA((2,)),
            pltpu.SemaphoreType.DMA((2,)),
        ],
        compiler_params=pltpu.CompilerParams(
            use_tc_tiling_on_sc=False,
            needs_layout_passes=False,
            skip_device_barrier=True,
        ),
    )
    def emb(idx_hbm, table_hbm, out_hbm, idx_v, rows_v, zpos_v, gsem, ssem):
        wid = lax.axis_index("s") * NC + lax.axis_index("c")
        pltpu.sync_copy(idx_hbm.at[wid], idx_v)
        base = wid * bpw

        def start_gather(c):
            b = c % 2
            for j in range(GPC):
                pltpu.async_copy(
                    table_hbm.at[idx_v.at[c * GPC + j]],
                    rows_v.at[b, pl.ds(j * GROUP, GROUP)],
                    gsem.at[b],
                )

        def wait_gather(c):
            b = c % 2
            for j in range(GPC):
                pltpu.make_async_copy(
                    table_hbm.at[idx_v.at[c * GPC + j]],
                    rows_v.at[b, pl.ds(j * GROUP, GROUP)],
                    gsem.at[b],
                ).wait()

        def fix_zeros(c):
            b = c % 2
            cnt = jnp.int32(0)
            for s in range(CHUNK // LANES):
                idxs = idx_v[c * GPC + s // (GROUP // LANES),
                             pl.ds((s % (GROUP // LANES)) * LANES, LANES)]
                m0 = idxs == 0
                pos16 = lax.iota(jnp.int32, LANES) + (s * LANES)
                plsc.store_compressed(zpos_v.at[pl.ds(cnt, LANES)], pos16, mask=m0)
                cnt = cnt + jnp.sum(m0.astype(jnp.int32))

            zeros = jnp.zeros((LANES,), jnp.float32)

            def zero_one(i, carry):
                r = jnp.max(plsc.load_gather(zpos_v, [jnp.full((LANES,), i, jnp.int32)]))
                for a in range(DIM // LANES):
                    rows_v[b, r, pl.ds(a * LANES, LANES)] = zeros
                return carry

            lax.fori_loop(0, cnt, zero_one, jnp.int32(0))

        def start_store(c):
            b = c % 2
            pltpu.async_copy(
                rows_v.at[b], out_hbm.at[pl.ds(base + c * CHUNK, CHUNK)], ssem.at[b]
            )

        def wait_store(c):
            b = c % 2
            pltpu.make_async_copy(
                rows_v.at[b], out_hbm.at[pl.ds(base + c * CHUNK, CHUNK)], ssem.at[b]
            ).wait()

        start_gather(0)
        for c in range(NCHUNK):
            wait_gather(c)
            fix_zeros(c)
            if c + 1 < NCHUNK:
                if c >= 1:
                    wait_store(c - 1)
                start_gather(c + 1)
            start_store(c)
        if NCHUNK >= 2:
            wait_store(NCHUNK - 2)
        wait_store(NCHUNK - 1)

    return emb


def kernel(input, table):
    batch, width = input.shape
    B = batch * width
    idx = input.T.reshape(NW, (B // NW) // GROUP, GROUP)
    out = _make_kernel(B)(idx, table)
    return out.reshape(width, batch, DIM).transpose(1, 0, 2)
